# baseline (speedup 1.0000x reference)
import functools

import jax
import jax.numpy as jnp
from jax import lax
from jax.experimental import pallas as pl
from jax.experimental.pallas import tpu as pltpu
from jax.experimental.pallas import tpu_sc as plsc

_VOCAB = 3875
_OFF4, _OFF5 = 125, 750

_B, _L = 4096, 1024
_NUM_TILES = 32
_ROWS_PER_TILE = _B // _NUM_TILES
_GROUP = 8
_N_GROUPS = _ROWS_PER_TILE // _GROUP
_SEQ_STRIDE = _L + 16


def _make_sc_kernel():
    mesh = plsc.VectorSubcoreMesh(core_axis_name="c", subcore_axis_name="s")

    @functools.partial(
        pl.kernel,
        out_type=jax.ShapeDtypeStruct((_B, _VOCAB), jnp.float32),
        mesh=mesh,
        compiler_params=pltpu.CompilerParams(needs_layout_passes=False),
        scratch_types=[
            pltpu.VMEM((_GROUP, _L), jnp.int32),
            pltpu.VMEM((_GROUP, _L), jnp.int32),
            pltpu.VMEM((_GROUP, _VOCAB), jnp.float32),
            pltpu.VMEM((_GROUP, _VOCAB), jnp.float32),
            pltpu.VMEM((_SEQ_STRIDE,), jnp.int32),
            pltpu.SemaphoreType.DMA,
            pltpu.SemaphoreType.DMA,
            pltpu.SemaphoreType.DMA,
            pltpu.SemaphoreType.DMA,
        ],
    )
    def ngram_sc(seq_hbm, out_hbm, seq_a, seq_b, buf_a, buf_b, strip,
                 in_sem_a, in_sem_b, out_sem_a, out_sem_b):
        wid = lax.axis_index("s") * 2 + lax.axis_index("c")
        tile_row0 = wid * _ROWS_PER_TILE
        zeros16f = jnp.zeros((16,), jnp.float32)
        ones16f = jnp.full((16,), 1.0, jnp.float32)
        zeros16i = jnp.zeros((16,), jnp.int32)
        lane = lax.iota(jnp.int32, 16)
        mask3 = lane <= 13
        mask4 = lane <= 12
        mask5 = lane <= 11

        seqs = (seq_a, seq_b)
        bufs = (buf_a, buf_b)
        in_sems = (in_sem_a, in_sem_b)
        out_sems = (out_sem_a, out_sem_b)

        def in_copy(g, p):
            row0 = tile_row0 + g * _GROUP
            return pltpu.make_async_copy(
                seq_hbm.at[pl.ds(row0, _GROUP)], seqs[p], in_sems[p])

        def out_copy(g, p):
            row0 = tile_row0 + g * _GROUP
            return pltpu.make_async_copy(
                bufs[p], out_hbm.at[pl.ds(row0, _GROUP)], out_sems[p])

        strip[pl.ds(_L, 16)] = zeros16i

        in_copy(0, 0).start()
        in_copy(1, 1).start()

        for g in range(_N_GROUPS):
            p = g % 2
            seq_v, buf = seqs[p], bufs[p]

            if g >= 2:
                out_copy(g - 2, p).wait()

            in_copy(g, p).wait()

            def do_row(r, carry):
                rv = jnp.full((16,), 1, jnp.int32) * r

                @plsc.parallel_loop(0, _L, step=16, unroll=8)
                def _stage(k):
                    strip[pl.ds(k, 16)] = seq_v[r, pl.ds(k, 16)]

                @plsc.parallel_loop(0, 3872, step=16, unroll=11)
                def _zero(k):
                    buf[r, pl.ds(k, 16)] = zeros16f

                buf[r, pl.ds(_VOCAB - 16, 16)] = zeros16f

                def codes_at(base):
                    v0 = strip[pl.ds(base, 16)]
                    v1 = strip[pl.ds(base + 1, 16)]
                    v2 = strip[pl.ds(base + 2, 16)]
                    v3 = strip[pl.ds(base + 3, 16)]
                    v4 = strip[pl.ds(base + 4, 16)]
                    c3 = (v0 * 5 + v1) * 5 + v2
                    c4 = c3 * 5 + v3
                    c5 = c4 * 5 + v4
                    return c3, c4, c5

                @plsc.parallel_loop(0, 63 * 16, step=16, unroll=4)
                def _chunk(i):
                    c3, c4, c5 = codes_at(i)
                    plsc.store_scatter(buf, [rv, c3], ones16f)
                    plsc.store_scatter(buf, [rv, c4 + _OFF4], ones16f)
                    plsc.store_scatter(buf, [rv, c5 + _OFF5], ones16f)

                c3, c4, c5 = codes_at(63 * 16)
                plsc.store_scatter(buf, [rv, c3], ones16f, mask=mask3)
                plsc.store_scatter(buf, [rv, c4 + _OFF4], ones16f, mask=mask4)
                plsc.store_scatter(buf, [rv, c5 + _OFF5], ones16f, mask=mask5)
                return carry

            lax.fori_loop(0, _GROUP, do_row, 0)

            out_copy(g, p).start()
            if g + 2 < _N_GROUPS:
                in_copy(g + 2, p).start()

        out_copy(_N_GROUPS - 2, 0).wait()
        out_copy(_N_GROUPS - 1, 1).wait()

    return ngram_sc


_NGRAM_SC = _make_sc_kernel()


@jax.jit
def kernel(seq):
    return _NGRAM_SC(seq)


# speedup vs baseline: 1.0403x; 1.0403x over previous
import functools

import jax
import jax.numpy as jnp
from jax import lax
from jax.experimental import pallas as pl
from jax.experimental.pallas import tpu as pltpu
from jax.experimental.pallas import tpu_sc as plsc

_VOCAB = 3875
_OFF4, _OFF5 = 125, 750

_B, _L = 4096, 1024
_NUM_TILES = 32
_ROWS_PER_TILE = _B // _NUM_TILES
_GROUP = 8
_N_GROUPS = _ROWS_PER_TILE // _GROUP
_SEQ_STRIDE = _L + 16


def _make_sc_kernel():
    mesh = plsc.VectorSubcoreMesh(core_axis_name="c", subcore_axis_name="s")

    @functools.partial(
        pl.kernel,
        out_type=jax.ShapeDtypeStruct((_B, _VOCAB), jnp.float32),
        mesh=mesh,
        compiler_params=pltpu.CompilerParams(needs_layout_passes=False),
        scratch_types=[
            pltpu.VMEM((_GROUP, _L), jnp.int32),
            pltpu.VMEM((_GROUP, _L), jnp.int32),
            pltpu.VMEM((_GROUP, _VOCAB), jnp.float32),
            pltpu.VMEM((_GROUP, _VOCAB), jnp.float32),
            pltpu.VMEM((_SEQ_STRIDE,), jnp.int32),
            pltpu.SemaphoreType.DMA,
            pltpu.SemaphoreType.DMA,
            pltpu.SemaphoreType.DMA,
            pltpu.SemaphoreType.DMA,
        ],
    )
    def ngram_sc(seq_hbm, out_hbm, seq_a, seq_b, buf_a, buf_b, strip,
                 in_sem_a, in_sem_b, out_sem_a, out_sem_b):
        wid = lax.axis_index("s") * 2 + lax.axis_index("c")
        tile_row0 = wid * _ROWS_PER_TILE
        zeros16f = jnp.zeros((16,), jnp.float32)
        ones16f = jnp.full((16,), 1.0, jnp.float32)
        zeros16i = jnp.zeros((16,), jnp.int32)
        lane = lax.iota(jnp.int32, 16)
        mask3 = lane <= 13
        mask4 = lane <= 12
        mask5 = lane <= 11

        seqs = (seq_a, seq_b)
        bufs = (buf_a, buf_b)
        in_sems = (in_sem_a, in_sem_b)
        out_sems = (out_sem_a, out_sem_b)

        def in_copy(g, p):
            row0 = tile_row0 + g * _GROUP
            return pltpu.make_async_copy(
                seq_hbm.at[pl.ds(row0, _GROUP)], seqs[p], in_sems[p])

        def out_copy(g, p):
            row0 = tile_row0 + g * _GROUP
            return pltpu.make_async_copy(
                bufs[p], out_hbm.at[pl.ds(row0, _GROUP)], out_sems[p])

        strip[pl.ds(_L, 16)] = zeros16i

        in_copy(0, 0).start()
        in_copy(1, 1).start()

        for g in range(_N_GROUPS):
            p = g % 2
            seq_v, buf = seqs[p], bufs[p]

            if g >= 2:
                out_copy(g - 2, p).wait()

            in_copy(g, p).wait()

            def do_row(r, carry):
                rv = jnp.full((16,), 1, jnp.int32) * r

                @plsc.parallel_loop(0, _L, step=16, unroll=8)
                def _stage(k):
                    strip[pl.ds(k, 16)] = seq_v[r, pl.ds(k, 16)]

                @plsc.parallel_loop(0, 3872, step=16, unroll=11)
                def _zero(k):
                    buf[r, pl.ds(k, 16)] = zeros16f

                buf[r, pl.ds(_VOCAB - 16, 16)] = zeros16f

                def codes_at(base):
                    v0 = strip[pl.ds(base, 16)]
                    v1 = strip[pl.ds(base + 1, 16)]
                    v2 = strip[pl.ds(base + 2, 16)]
                    v3 = strip[pl.ds(base + 3, 16)]
                    v4 = strip[pl.ds(base + 4, 16)]
                    c3 = (v0 * 5 + v1) * 5 + v2
                    c4 = c3 * 5 + v3
                    c5 = c4 * 5 + v4
                    return c3, c4, c5

                @plsc.parallel_loop(0, 63 * 16, step=16, unroll=3)
                def _chunk(i):
                    c3, c4, c5 = codes_at(i)
                    plsc.store_scatter(buf, [rv, c3], ones16f)
                    plsc.store_scatter(buf, [rv, c4 + _OFF4], ones16f)
                    plsc.store_scatter(buf, [rv, c5 + _OFF5], ones16f)

                c3, c4, c5 = codes_at(63 * 16)
                plsc.store_scatter(buf, [rv, c3], ones16f, mask=mask3)
                plsc.store_scatter(buf, [rv, c4 + _OFF4], ones16f, mask=mask4)
                plsc.store_scatter(buf, [rv, c5 + _OFF5], ones16f, mask=mask5)
                return carry

            lax.fori_loop(0, _GROUP, do_row, 0)

            out_copy(g, p).start()
            if g + 2 < _N_GROUPS:
                in_copy(g + 2, p).start()

        out_copy(_N_GROUPS - 2, 0).wait()
        out_copy(_N_GROUPS - 1, 1).wait()

    return ngram_sc


_NGRAM_SC = _make_sc_kernel()


@jax.jit
def kernel(seq):
    return _NGRAM_SC(seq)
